# baseline (speedup 1.0000x reference)
import functools

import jax
import jax.numpy as jnp
from jax import lax
from jax.experimental import pallas as pl
from jax.experimental.pallas import tpu as pltpu
from jax.experimental.pallas import tpu_sc as plsc

N_NODES = 10000
N_EDGES = 320000
IN_CH = 128
OUT_CH = 128
HID = 256

NC = 2
NS = 16
L = 16

CHUNK = 128
E_TILE = N_EDGES // NS
FULLR = E_TILE // CHUNK
TAIL = E_TILE - FULLR * CHUNK
NCH = 160
PRE_ROWS = NS * NCH
HALF = N_NODES // NC
ACC_ROWS = 5120
RPT = ACC_ROWS // NS
TRASH = N_NODES
LTRASH = HALF

_mesh = plsc.VectorSubcoreMesh(core_axis_name="c", subcore_axis_name="s")


@functools.partial(
    pl.kernel,
    out_type=(
        jax.ShapeDtypeStruct((NC, ACC_ROWS, L), jnp.float32),
        jax.ShapeDtypeStruct((PRE_ROWS, CHUNK), jnp.int32),
        jax.ShapeDtypeStruct((NC, PRE_ROWS, CHUNK), jnp.int32),
    ),
    mesh=_mesh,
    scratch_types=[
        pltpu.VMEM((NCH, CHUNK), jnp.int32),
        pltpu.VMEM((CHUNK, L), jnp.float32),
        pltpu.VMEM((L, L), jnp.float32),
        pltpu.VMEM_SHARED((ACC_ROWS, L), jnp.float32),
    ],
)
def _prep_kernel(ei_hbm, deg_hbm, srcp_hbm, dstp_hbm, e2v, ones_v, zb_v, acc_sh):
    c = lax.axis_index("c")
    s = lax.axis_index("s")

    zrow = jnp.zeros((L,), jnp.float32)
    onerow = jnp.where(lax.iota(jnp.int32, L) == 0, 1.0, 0.0)
    for i in range(L):
        zb_v[i] = zrow
    for i in range(CHUNK):
        ones_v[i] = onerow

    base = s * RPT
    for k in range(RPT // L):
        pltpu.sync_copy(zb_v, acc_sh.at[pl.ds(base + k * L, L)])

    def stage(row, fill):
        ebase = row * N_EDGES + s * E_TILE
        for r in range(FULLR):
            pltpu.sync_copy(ei_hbm.at[pl.ds(ebase + r * CHUNK, CHUNK)],
                            e2v.at[r])
        pltpu.sync_copy(ei_hbm.at[pl.ds(ebase + FULLR * CHUNK, TAIL)],
                        e2v.at[FULLR, pl.ds(0, TAIL)])
        fill_vec = jnp.full((L,), fill, jnp.int32)
        for t in range(TAIL // L, CHUNK // L):
            e2v[FULLR, pl.ds(t * L, L)] = fill_vec
        for r in range(FULLR + 1, NCH):
            for t in range(CHUNK // L):
                e2v[r, pl.ds(t * L, L)] = fill_vec

    prow = pl.ds(s * NCH, NCH)

    @pl.when(c == 0)
    def _():
        stage(0, 0)
        pltpu.sync_copy(e2v, srcp_hbm.at[prow])

    stage(1, TRASH)
    cbase = c * HALF

    def remap(j, carry):
        for q in range(CHUNK // L):
            d = e2v[j, pl.ds(q * L, L)]
            loc = d - cbase
            ok = (loc >= 0) & (loc < HALF)
            e2v[j, pl.ds(q * L, L)] = jnp.where(ok, loc, LTRASH)
        return carry

    lax.fori_loop(0, NCH, remap, 0)
    pltpu.sync_copy(e2v, dstp_hbm.at[c, prow])
    plsc.subcore_barrier()

    def step(j, carry):
        pltpu.sync_copy(ones_v, acc_sh.at[e2v.at[j]], add=True)
        return carry

    lax.fori_loop(0, NCH, step, 0)
    plsc.subcore_barrier()

    pltpu.sync_copy(acc_sh.at[pl.ds(base, RPT)],
                    deg_hbm.at[c, pl.ds(base, RPT)])


@functools.partial(
    pl.kernel,
    out_type=jax.ShapeDtypeStruct((NC, ACC_ROWS, IN_CH), jnp.float32),
    mesh=_mesh,
    scratch_types=[
        pltpu.VMEM((NCH, CHUNK), jnp.int32),
        pltpu.VMEM((NCH, CHUNK), jnp.int32),
        pltpu.VMEM((CHUNK, IN_CH), jnp.float32),
        pltpu.VMEM((CHUNK, IN_CH), jnp.float32),
        pltpu.VMEM((L, IN_CH), jnp.float32),
        pltpu.VMEM_SHARED((ACC_ROWS, IN_CH), jnp.float32),
        pltpu.SemaphoreType.DMA,
        pltpu.SemaphoreType.DMA,
        pltpu.SemaphoreType.DMA,
        pltpu.SemaphoreType.DMA,
    ],
)
def _agg_kernel(tab_hbm, srcp_hbm, dstp_hbm, out_hbm,
                src_v, dst_v, buf0, buf1, zb_v, acc_sh,
                gsem0, gsem1, ssem0, ssem1):
    c = lax.axis_index("c")
    s = lax.axis_index("s")

    zrow = jnp.zeros((L,), jnp.float32)
    for i in range(L):
        for q in range(IN_CH // L):
            zb_v[i, pl.ds(q * L, L)] = zrow

    base = s * RPT
    for k in range(RPT // L):
        pltpu.sync_copy(zb_v, acc_sh.at[pl.ds(base + k * L, L)])

    prow = pl.ds(s * NCH, NCH)
    pltpu.sync_copy(srcp_hbm.at[prow], src_v)
    pltpu.sync_copy(dstp_hbm.at[c, prow], dst_v)
    plsc.subcore_barrier()

    pltpu.async_copy(tab_hbm.at[src_v.at[0]], buf0, gsem0)

    def step(g, carry):
        for b, (buf, gsem, ssem, nbuf, ngsem, nssem) in enumerate((
                (buf0, gsem0, ssem0, buf1, gsem1, ssem1),
                (buf1, gsem1, ssem1, buf0, gsem0, ssem0))):
            j = g * 2 + b
            pltpu.make_async_copy(tab_hbm.at[src_v.at[j]], buf, gsem).wait()

            @pl.when(j >= 1)
            def _():
                pltpu.make_async_copy(nbuf, acc_sh.at[dst_v.at[j]],
                                      nssem).wait()

            @pl.when(j < NCH - 1)
            def _():
                pltpu.async_copy(tab_hbm.at[src_v.at[j + 1]], nbuf, ngsem)

            pltpu.async_copy(buf, acc_sh.at[dst_v.at[j]], ssem, add=True)
        return carry

    lax.fori_loop(0, NCH // 2, step, 0)
    pltpu.make_async_copy(buf1, acc_sh.at[dst_v.at[NCH - 1]], ssem1).wait()
    plsc.subcore_barrier()

    pltpu.sync_copy(acc_sh.at[pl.ds(base, RPT)],
                    out_hbm.at[c, pl.ds(base, RPT)])


_ROWB = 1000
_GRID = N_NODES // _ROWB
_CBLK = _GRID // NC


def _hspec(width):
    return pl.BlockSpec((1, _ROWB, width), lambda i: (i // _CBLK, i % _CBLK, 0))


def _prescale_body(d_ref, x_ref, dinv_ref, xs_ref):
    deg = 1.0 + d_ref[0]
    dinv = lax.rsqrt(deg)
    dinv_ref[...] = dinv
    xs_ref[...] = x_ref[...] * dinv


def _prescale(d, x):
    return pl.pallas_call(
        _prescale_body,
        grid=(_GRID,),
        in_specs=[
            _hspec(1),
            pl.BlockSpec((_ROWB, IN_CH), lambda i: (i, 0)),
        ],
        out_specs=[
            pl.BlockSpec((_ROWB, 1), lambda i: (i, 0)),
            pl.BlockSpec((_ROWB, IN_CH), lambda i: (i, 0)),
        ],
        out_shape=[
            jax.ShapeDtypeStruct((N_NODES, 1), jnp.float32),
            jax.ShapeDtypeStruct((N_NODES, IN_CH), jnp.float32),
        ],
    )(d, x)


def _mid_body(p_ref, x_ref, dinv_ref, w1_ref, b1_ref, w2_ref,
              g_ref, gs_ref):
    dv = dinv_ref[...]
    t = dv * p_ref[0] + (dv * dv) * x_ref[...]
    h = jnp.dot(t, w1_ref[...], preferred_element_type=jnp.float32)
    h = jnp.maximum(h + b1_ref[...], 0.0)
    g = jnp.dot(h, w2_ref[...], preferred_element_type=jnp.float32)
    g_ref[...] = g
    gs_ref[...] = dv * g


def _mid(p, x, dinv, w1, b1, w2):
    return pl.pallas_call(
        _mid_body,
        grid=(_GRID,),
        in_specs=[
            _hspec(IN_CH),
            pl.BlockSpec((_ROWB, IN_CH), lambda i: (i, 0)),
            pl.BlockSpec((_ROWB, 1), lambda i: (i, 0)),
            pl.BlockSpec((IN_CH, HID), lambda i: (0, 0)),
            pl.BlockSpec((1, HID), lambda i: (0, 0)),
            pl.BlockSpec((HID, OUT_CH), lambda i: (0, 0)),
        ],
        out_specs=[
            pl.BlockSpec((_ROWB, OUT_CH), lambda i: (i, 0)),
            pl.BlockSpec((_ROWB, OUT_CH), lambda i: (i, 0)),
        ],
        out_shape=[
            jax.ShapeDtypeStruct((N_NODES, OUT_CH), jnp.float32),
            jax.ShapeDtypeStruct((N_NODES, OUT_CH), jnp.float32),
        ],
    )(p, x, dinv, w1, b1, w2)


def _final_body(q_ref, g_ref, dinv_ref, b2_ref, out_ref):
    dv = dinv_ref[...]
    out_ref[...] = dv * q_ref[0] + (dv * dv) * g_ref[...] + b2_ref[...]


def _final(q, g, dinv, b2):
    return pl.pallas_call(
        _final_body,
        grid=(_GRID,),
        in_specs=[
            _hspec(OUT_CH),
            pl.BlockSpec((_ROWB, OUT_CH), lambda i: (i, 0)),
            pl.BlockSpec((_ROWB, 1), lambda i: (i, 0)),
            pl.BlockSpec((1, OUT_CH), lambda i: (0, 0)),
        ],
        out_specs=pl.BlockSpec((_ROWB, OUT_CH), lambda i: (i, 0)),
        out_shape=jax.ShapeDtypeStruct((N_NODES, OUT_CH), jnp.float32),
    )(q, g, dinv, b2)


def kernel(x, edge_index, W1, b1, W2, b2):
    ei = edge_index.astype(jnp.int32).reshape(2 * N_EDGES)

    degp, srcp, dstp = _prep_kernel(ei)
    d = degp[:, :HALF, 0:1]

    dinv, xs = _prescale(d, x)

    p = _agg_kernel(xs, srcp, dstp)
    g, gs = _mid(p[:, :HALF], x, dinv, W1, b1.reshape(1, HID), W2)

    q = _agg_kernel(gs, srcp, dstp)
    return _final(q[:, :HALF], g, dinv, b2.reshape(1, OUT_CH))


# speedup vs baseline: 1.0740x; 1.0740x over previous
import functools

import jax
import jax.numpy as jnp
from jax import lax
from jax.experimental import pallas as pl
from jax.experimental.pallas import tpu as pltpu
from jax.experimental.pallas import tpu_sc as plsc

N_NODES = 10000
N_EDGES = 320000
IN_CH = 128
OUT_CH = 128
HID = 256

NC = 2
NS = 16
L = 16

CHUNK = 128
E_TILE = N_EDGES // NS
FULLR = E_TILE // CHUNK
TAIL = E_TILE - FULLR * CHUNK
NCH = 160
PRE_ROWS = NS * NCH
HALF = N_NODES // NC
ACC_ROWS = 5120
RPT = ACC_ROWS // NS
TRASH = N_NODES
LTRASH = HALF

_mesh = plsc.VectorSubcoreMesh(core_axis_name="c", subcore_axis_name="s")


@functools.partial(
    pl.kernel,
    out_type=(
        jax.ShapeDtypeStruct((NC, ACC_ROWS, L), jnp.float32),
        jax.ShapeDtypeStruct((PRE_ROWS, CHUNK), jnp.int32),
        jax.ShapeDtypeStruct((NC, PRE_ROWS, CHUNK), jnp.int32),
    ),
    mesh=_mesh,
    scratch_types=[
        pltpu.VMEM((NCH, CHUNK), jnp.int32),
        pltpu.VMEM((CHUNK, L), jnp.float32),
        pltpu.VMEM((L, L), jnp.float32),
        pltpu.VMEM_SHARED((ACC_ROWS, L), jnp.float32),
        pltpu.SemaphoreType.DMA,
    ],
)
def _prep_kernel(ei_hbm, deg_hbm, srcp_hbm, dstp_hbm,
                 e2v, ones_v, zb_v, acc_sh, stsem):
    c = lax.axis_index("c")
    s = lax.axis_index("s")

    zrow = jnp.zeros((L,), jnp.float32)
    onerow = jnp.where(lax.iota(jnp.int32, L) == 0, 1.0, 0.0)
    for i in range(L):
        zb_v[i] = zrow
    for i in range(CHUNK):
        ones_v[i] = onerow

    base = s * RPT
    for k in range(RPT // L):
        pltpu.sync_copy(zb_v, acc_sh.at[pl.ds(base + k * L, L)])

    def stage(row, fill):
        ebase = row * N_EDGES + s * E_TILE
        for r in range(FULLR):
            pltpu.async_copy(ei_hbm.at[pl.ds(ebase + r * CHUNK, CHUNK)],
                             e2v.at[r], stsem)
        pltpu.async_copy(ei_hbm.at[pl.ds(ebase + FULLR * CHUNK, TAIL)],
                         e2v.at[FULLR, pl.ds(0, TAIL)], stsem)
        for r in range(FULLR):
            pltpu.make_async_copy(ei_hbm.at[pl.ds(ebase + r * CHUNK, CHUNK)],
                                  e2v.at[r], stsem).wait()
        pltpu.make_async_copy(ei_hbm.at[pl.ds(ebase + FULLR * CHUNK, TAIL)],
                              e2v.at[FULLR, pl.ds(0, TAIL)], stsem).wait()
        fill_vec = jnp.full((L,), fill, jnp.int32)
        for t in range(TAIL // L, CHUNK // L):
            e2v[FULLR, pl.ds(t * L, L)] = fill_vec
        for r in range(FULLR + 1, NCH):
            for t in range(CHUNK // L):
                e2v[r, pl.ds(t * L, L)] = fill_vec

    prow = pl.ds(s * NCH, NCH)

    @pl.when(c == 0)
    def _():
        stage(0, 0)
        pltpu.sync_copy(e2v, srcp_hbm.at[prow])

    stage(1, TRASH)
    cbase = c * HALF

    def remap(j, carry):
        for q in range(CHUNK // L):
            d = e2v[j, pl.ds(q * L, L)]
            loc = d - cbase
            ok = (loc >= 0) & (loc < HALF)
            e2v[j, pl.ds(q * L, L)] = jnp.where(ok, loc, LTRASH)
        return carry

    lax.fori_loop(0, NCH, remap, 0)
    pltpu.sync_copy(e2v, dstp_hbm.at[c, prow])
    plsc.subcore_barrier()

    def step(j, carry):
        pltpu.sync_copy(ones_v, acc_sh.at[e2v.at[j]], add=True)
        return carry

    lax.fori_loop(0, NCH, step, 0)
    plsc.subcore_barrier()

    pltpu.sync_copy(acc_sh.at[pl.ds(base, RPT)],
                    deg_hbm.at[c, pl.ds(base, RPT)])


@functools.partial(
    pl.kernel,
    out_type=jax.ShapeDtypeStruct((NC, ACC_ROWS, IN_CH), jnp.float32),
    mesh=_mesh,
    scratch_types=[
        pltpu.VMEM((NCH, CHUNK), jnp.int32),
        pltpu.VMEM((NCH, CHUNK), jnp.int32),
        pltpu.VMEM((CHUNK, IN_CH), jnp.float32),
        pltpu.VMEM((CHUNK, IN_CH), jnp.float32),
        pltpu.VMEM((L, IN_CH), jnp.float32),
        pltpu.VMEM_SHARED((ACC_ROWS, IN_CH), jnp.float32),
        pltpu.SemaphoreType.DMA,
        pltpu.SemaphoreType.DMA,
    ],
)
def _agg_kernel(tab_hbm, srcp_hbm, dstp_hbm, out_hbm,
                src_v, dst_v, buf0, buf1, zb_v, acc_sh, gsem0, gsem1):
    c = lax.axis_index("c")
    s = lax.axis_index("s")

    zrow = jnp.zeros((L,), jnp.float32)
    for i in range(L):
        for q in range(IN_CH // L):
            zb_v[i, pl.ds(q * L, L)] = zrow

    base = s * RPT
    for k in range(RPT // L):
        pltpu.sync_copy(zb_v, acc_sh.at[pl.ds(base + k * L, L)])

    prow = pl.ds(s * NCH, NCH)
    pltpu.sync_copy(srcp_hbm.at[prow], src_v)
    pltpu.sync_copy(dstp_hbm.at[c, prow], dst_v)
    plsc.subcore_barrier()

    pltpu.async_copy(tab_hbm.at[src_v.at[0]], buf0, gsem0)

    def step(g, carry):
        for b, (buf, gsem, nbuf, ngsem) in enumerate((
                (buf0, gsem0, buf1, gsem1),
                (buf1, gsem1, buf0, gsem0))):
            j = g * 2 + b
            pltpu.make_async_copy(tab_hbm.at[src_v.at[j]], buf, gsem).wait()

            @pl.when(j < NCH - 1)
            def _():
                pltpu.async_copy(tab_hbm.at[src_v.at[j + 1]], nbuf, ngsem)

            pltpu.sync_copy(buf, acc_sh.at[dst_v.at[j]], add=True)
        return carry

    lax.fori_loop(0, NCH // 2, step, 0)
    plsc.subcore_barrier()

    pltpu.sync_copy(acc_sh.at[pl.ds(base, RPT)],
                    out_hbm.at[c, pl.ds(base, RPT)])


_ROWB = 1000
_GRID = N_NODES // _ROWB
_CBLK = _GRID // NC


def _hspec(width):
    return pl.BlockSpec((1, _ROWB, width), lambda i: (i // _CBLK, i % _CBLK, 0))


def _prescale_body(d_ref, x_ref, dinv_ref, xs_ref):
    deg = 1.0 + d_ref[0]
    dinv = lax.rsqrt(deg)
    dinv_ref[...] = dinv
    xs_ref[...] = x_ref[...] * dinv


def _prescale(d, x):
    return pl.pallas_call(
        _prescale_body,
        grid=(_GRID,),
        in_specs=[
            _hspec(1),
            pl.BlockSpec((_ROWB, IN_CH), lambda i: (i, 0)),
        ],
        out_specs=[
            pl.BlockSpec((_ROWB, 1), lambda i: (i, 0)),
            pl.BlockSpec((_ROWB, IN_CH), lambda i: (i, 0)),
        ],
        out_shape=[
            jax.ShapeDtypeStruct((N_NODES, 1), jnp.float32),
            jax.ShapeDtypeStruct((N_NODES, IN_CH), jnp.float32),
        ],
    )(d, x)


def _mid_body(p_ref, x_ref, dinv_ref, w1_ref, b1_ref, w2_ref,
              g_ref, gs_ref):
    dv = dinv_ref[...]
    t = dv * p_ref[0] + (dv * dv) * x_ref[...]
    h = jnp.dot(t, w1_ref[...], preferred_element_type=jnp.float32)
    h = jnp.maximum(h + b1_ref[...], 0.0)
    g = jnp.dot(h, w2_ref[...], preferred_element_type=jnp.float32)
    g_ref[...] = g
    gs_ref[...] = dv * g


def _mid(p, x, dinv, w1, b1, w2):
    return pl.pallas_call(
        _mid_body,
        grid=(_GRID,),
        in_specs=[
            _hspec(IN_CH),
            pl.BlockSpec((_ROWB, IN_CH), lambda i: (i, 0)),
            pl.BlockSpec((_ROWB, 1), lambda i: (i, 0)),
            pl.BlockSpec((IN_CH, HID), lambda i: (0, 0)),
            pl.BlockSpec((1, HID), lambda i: (0, 0)),
            pl.BlockSpec((HID, OUT_CH), lambda i: (0, 0)),
        ],
        out_specs=[
            pl.BlockSpec((_ROWB, OUT_CH), lambda i: (i, 0)),
            pl.BlockSpec((_ROWB, OUT_CH), lambda i: (i, 0)),
        ],
        out_shape=[
            jax.ShapeDtypeStruct((N_NODES, OUT_CH), jnp.float32),
            jax.ShapeDtypeStruct((N_NODES, OUT_CH), jnp.float32),
        ],
    )(p, x, dinv, w1, b1, w2)


def _final_body(q_ref, g_ref, dinv_ref, b2_ref, out_ref):
    dv = dinv_ref[...]
    out_ref[...] = dv * q_ref[0] + (dv * dv) * g_ref[...] + b2_ref[...]


def _final(q, g, dinv, b2):
    return pl.pallas_call(
        _final_body,
        grid=(_GRID,),
        in_specs=[
            _hspec(OUT_CH),
            pl.BlockSpec((_ROWB, OUT_CH), lambda i: (i, 0)),
            pl.BlockSpec((_ROWB, 1), lambda i: (i, 0)),
            pl.BlockSpec((1, OUT_CH), lambda i: (0, 0)),
        ],
        out_specs=pl.BlockSpec((_ROWB, OUT_CH), lambda i: (i, 0)),
        out_shape=jax.ShapeDtypeStruct((N_NODES, OUT_CH), jnp.float32),
    )(q, g, dinv, b2)


def kernel(x, edge_index, W1, b1, W2, b2):
    ei = edge_index.astype(jnp.int32).reshape(2 * N_EDGES)

    degp, srcp, dstp = _prep_kernel(ei)
    d = degp[:, :HALF, 0:1]

    dinv, xs = _prescale(d, x)

    p = _agg_kernel(xs, srcp, dstp)
    g, gs = _mid(p[:, :HALF], x, dinv, W1, b1.reshape(1, HID), W2)

    q = _agg_kernel(gs, srcp, dstp)
    return _final(q[:, :HALF], g, dinv, b2.reshape(1, OUT_CH))


# speedup vs baseline: 1.1942x; 1.1119x over previous
import functools

import jax
import jax.numpy as jnp
from jax import lax
from jax.experimental import pallas as pl
from jax.experimental.pallas import tpu as pltpu
from jax.experimental.pallas import tpu_sc as plsc

N_NODES = 10000
N_EDGES = 320000
IN_CH = 128
OUT_CH = 128
HID = 256

NC = 2
NS = 16
L = 16

CHUNK = 128
E_TILE = N_EDGES // NS
FULLR = E_TILE // CHUNK
TAIL = E_TILE - FULLR * CHUNK
NCH = 160
PRE_ROWS = NS * NCH
HALF = N_NODES // NC
ACC_ROWS = 5120
RPT = ACC_ROWS // NS
TRASH = N_NODES
LTRASH = HALF

_mesh = plsc.VectorSubcoreMesh(core_axis_name="c", subcore_axis_name="s")


@functools.partial(
    pl.kernel,
    out_type=(
        jax.ShapeDtypeStruct((NC, ACC_ROWS, L), jnp.float32),
        jax.ShapeDtypeStruct((PRE_ROWS, CHUNK), jnp.int32),
        jax.ShapeDtypeStruct((NC, PRE_ROWS, CHUNK), jnp.int32),
    ),
    mesh=_mesh,
    scratch_types=[
        pltpu.VMEM((NCH, CHUNK), jnp.int32),
        pltpu.VMEM((CHUNK, L), jnp.float32),
        pltpu.VMEM((L, L), jnp.float32),
        pltpu.VMEM_SHARED((ACC_ROWS, L), jnp.float32),
        pltpu.SemaphoreType.DMA,
    ],
)
def _prep_kernel(ei_hbm, deg_hbm, srcp_hbm, dstp_hbm,
                 e2v, ones_v, zb_v, acc_sh, stsem):
    c = lax.axis_index("c")
    s = lax.axis_index("s")

    zrow = jnp.zeros((L,), jnp.float32)
    onerow = jnp.where(lax.iota(jnp.int32, L) == 0, 1.0, 0.0)
    for i in range(L):
        zb_v[i] = zrow
    for i in range(CHUNK):
        ones_v[i] = onerow

    base = s * RPT
    for k in range(RPT // L):
        pltpu.sync_copy(zb_v, acc_sh.at[pl.ds(base + k * L, L)])

    def stage(row, fill):
        ebase = row * N_EDGES + s * E_TILE
        for r in range(FULLR):
            pltpu.async_copy(ei_hbm.at[pl.ds(ebase + r * CHUNK, CHUNK)],
                             e2v.at[r], stsem)
        pltpu.async_copy(ei_hbm.at[pl.ds(ebase + FULLR * CHUNK, TAIL)],
                         e2v.at[FULLR, pl.ds(0, TAIL)], stsem)
        for r in range(FULLR):
            pltpu.make_async_copy(ei_hbm.at[pl.ds(ebase + r * CHUNK, CHUNK)],
                                  e2v.at[r], stsem).wait()
        pltpu.make_async_copy(ei_hbm.at[pl.ds(ebase + FULLR * CHUNK, TAIL)],
                              e2v.at[FULLR, pl.ds(0, TAIL)], stsem).wait()
        fill_vec = jnp.full((L,), fill, jnp.int32)
        for t in range(TAIL // L, CHUNK // L):
            e2v[FULLR, pl.ds(t * L, L)] = fill_vec
        for r in range(FULLR + 1, NCH):
            for t in range(CHUNK // L):
                e2v[r, pl.ds(t * L, L)] = fill_vec

    prow = pl.ds(s * NCH, NCH)

    @pl.when(c == 0)
    def _():
        stage(0, 0)
        pltpu.sync_copy(e2v, srcp_hbm.at[prow])

    stage(1, TRASH)
    cbase = c * HALF

    def remap(j, carry):
        for q in range(CHUNK // L):
            d = e2v[j, pl.ds(q * L, L)]
            loc = d - cbase
            ok = (loc >= 0) & (loc < HALF)
            e2v[j, pl.ds(q * L, L)] = jnp.where(ok, loc, LTRASH + (d & 63))
        return carry

    lax.fori_loop(0, NCH, remap, 0)
    pltpu.sync_copy(e2v, dstp_hbm.at[c, prow])
    plsc.subcore_barrier()

    def step(j, carry):
        pltpu.sync_copy(ones_v, acc_sh.at[e2v.at[j]], add=True)
        return carry

    lax.fori_loop(0, NCH, step, 0)
    plsc.subcore_barrier()

    pltpu.sync_copy(acc_sh.at[pl.ds(base, RPT)],
                    deg_hbm.at[c, pl.ds(base, RPT)])


@functools.partial(
    pl.kernel,
    out_type=jax.ShapeDtypeStruct((NC, ACC_ROWS, IN_CH), jnp.float32),
    mesh=_mesh,
    scratch_types=[
        pltpu.VMEM((NCH, CHUNK), jnp.int32),
        pltpu.VMEM((NCH, CHUNK), jnp.int32),
        pltpu.VMEM((CHUNK, IN_CH), jnp.float32),
        pltpu.VMEM((CHUNK, IN_CH), jnp.float32),
        pltpu.VMEM((L, IN_CH), jnp.float32),
        pltpu.VMEM_SHARED((ACC_ROWS, IN_CH), jnp.float32),
        pltpu.SemaphoreType.DMA,
        pltpu.SemaphoreType.DMA,
    ],
)
def _agg_kernel(tab_hbm, srcp_hbm, dstp_hbm, out_hbm,
                src_v, dst_v, buf0, buf1, zb_v, acc_sh, gsem0, gsem1):
    c = lax.axis_index("c")
    s = lax.axis_index("s")

    zrow = jnp.zeros((L,), jnp.float32)
    for i in range(L):
        for q in range(IN_CH // L):
            zb_v[i, pl.ds(q * L, L)] = zrow

    base = s * RPT
    for k in range(RPT // L):
        pltpu.sync_copy(zb_v, acc_sh.at[pl.ds(base + k * L, L)])

    prow = pl.ds(s * NCH, NCH)
    pltpu.sync_copy(srcp_hbm.at[prow], src_v)
    pltpu.sync_copy(dstp_hbm.at[c, prow], dst_v)
    plsc.subcore_barrier()

    pltpu.async_copy(tab_hbm.at[src_v.at[0]], buf0, gsem0)

    def step(g, carry):
        for b, (buf, gsem, nbuf, ngsem) in enumerate((
                (buf0, gsem0, buf1, gsem1),
                (buf1, gsem1, buf0, gsem0))):
            j = g * 2 + b
            pltpu.make_async_copy(tab_hbm.at[src_v.at[j]], buf, gsem).wait()

            @pl.when(j < NCH - 1)
            def _():
                pltpu.async_copy(tab_hbm.at[src_v.at[j + 1]], nbuf, ngsem)

            pltpu.sync_copy(buf, acc_sh.at[dst_v.at[j]], add=True)
        return carry

    lax.fori_loop(0, NCH // 2, step, 0)
    plsc.subcore_barrier()

    pltpu.sync_copy(acc_sh.at[pl.ds(base, RPT)],
                    out_hbm.at[c, pl.ds(base, RPT)])


_ROWB = 1000
_GRID = N_NODES // _ROWB
_CBLK = _GRID // NC


def _hspec(width):
    return pl.BlockSpec((1, _ROWB, width), lambda i: (i // _CBLK, i % _CBLK, 0))


def _prescale_body(d_ref, x_ref, dinv_ref, xs_ref):
    deg = 1.0 + d_ref[0]
    dinv = lax.rsqrt(deg)
    dinv_ref[...] = dinv
    xs_ref[...] = x_ref[...] * dinv


def _prescale(d, x):
    return pl.pallas_call(
        _prescale_body,
        grid=(_GRID,),
        in_specs=[
            _hspec(1),
            pl.BlockSpec((_ROWB, IN_CH), lambda i: (i, 0)),
        ],
        out_specs=[
            pl.BlockSpec((_ROWB, 1), lambda i: (i, 0)),
            pl.BlockSpec((_ROWB, IN_CH), lambda i: (i, 0)),
        ],
        out_shape=[
            jax.ShapeDtypeStruct((N_NODES, 1), jnp.float32),
            jax.ShapeDtypeStruct((N_NODES, IN_CH), jnp.float32),
        ],
    )(d, x)


def _mid_body(p_ref, x_ref, dinv_ref, w1_ref, b1_ref, w2_ref,
              g_ref, gs_ref):
    dv = dinv_ref[...]
    t = dv * p_ref[0] + (dv * dv) * x_ref[...]
    h = jnp.dot(t, w1_ref[...], preferred_element_type=jnp.float32)
    h = jnp.maximum(h + b1_ref[...], 0.0)
    g = jnp.dot(h, w2_ref[...], preferred_element_type=jnp.float32)
    g_ref[...] = g
    gs_ref[...] = dv * g


def _mid(p, x, dinv, w1, b1, w2):
    return pl.pallas_call(
        _mid_body,
        grid=(_GRID,),
        in_specs=[
            _hspec(IN_CH),
            pl.BlockSpec((_ROWB, IN_CH), lambda i: (i, 0)),
            pl.BlockSpec((_ROWB, 1), lambda i: (i, 0)),
            pl.BlockSpec((IN_CH, HID), lambda i: (0, 0)),
            pl.BlockSpec((1, HID), lambda i: (0, 0)),
            pl.BlockSpec((HID, OUT_CH), lambda i: (0, 0)),
        ],
        out_specs=[
            pl.BlockSpec((_ROWB, OUT_CH), lambda i: (i, 0)),
            pl.BlockSpec((_ROWB, OUT_CH), lambda i: (i, 0)),
        ],
        out_shape=[
            jax.ShapeDtypeStruct((N_NODES, OUT_CH), jnp.float32),
            jax.ShapeDtypeStruct((N_NODES, OUT_CH), jnp.float32),
        ],
    )(p, x, dinv, w1, b1, w2)


def _final_body(q_ref, g_ref, dinv_ref, b2_ref, out_ref):
    dv = dinv_ref[...]
    out_ref[...] = dv * q_ref[0] + (dv * dv) * g_ref[...] + b2_ref[...]


def _final(q, g, dinv, b2):
    return pl.pallas_call(
        _final_body,
        grid=(_GRID,),
        in_specs=[
            _hspec(OUT_CH),
            pl.BlockSpec((_ROWB, OUT_CH), lambda i: (i, 0)),
            pl.BlockSpec((_ROWB, 1), lambda i: (i, 0)),
            pl.BlockSpec((1, OUT_CH), lambda i: (0, 0)),
        ],
        out_specs=pl.BlockSpec((_ROWB, OUT_CH), lambda i: (i, 0)),
        out_shape=jax.ShapeDtypeStruct((N_NODES, OUT_CH), jnp.float32),
    )(q, g, dinv, b2)


def kernel(x, edge_index, W1, b1, W2, b2):
    ei = edge_index.astype(jnp.int32).reshape(2 * N_EDGES)

    degp, srcp, dstp = _prep_kernel(ei)
    d = degp[:, :HALF, 0:1]

    dinv, xs = _prescale(d, x)

    p = _agg_kernel(xs, srcp, dstp)
    g, gs = _mid(p[:, :HALF], x, dinv, W1, b1.reshape(1, HID), W2)

    q = _agg_kernel(gs, srcp, dstp)
    return _final(q[:, :HALF], g, dinv, b2.reshape(1, OUT_CH))
